# baseline (speedup 1.0000x reference)
import functools

import jax
import jax.numpy as jnp
from jax import lax
from jax.experimental import pallas as pl
from jax.experimental.pallas import tpu as pltpu
from jax.experimental.pallas import tpu_sc as plsc

_BM = 256
_GATHER_CHUNK = 40
_COMBINE_CHUNK = 32


def _routing_body(logits_ref, sel_ref, tw_ref, *, n_experts):
    logits = logits_ref[...]
    m = jnp.max(logits, axis=1, keepdims=True)
    p = jnp.exp(logits - m)
    p = p / jnp.sum(p, axis=1, keepdims=True)
    lane = jax.lax.broadcasted_iota(jnp.int32, p.shape, 1)
    e1 = jnp.argmax(p, axis=1).astype(jnp.int32)
    p1 = jnp.max(p, axis=1)
    pm = jnp.where(lane == e1[:, None], -jnp.inf, p)
    e2 = jnp.argmax(pm, axis=1).astype(jnp.int32)
    p2 = jnp.max(pm, axis=1)
    denom = p1 + p2
    sel_ref[...] = jnp.concatenate([e1[:, None], e2[:, None]], axis=1)
    tw_ref[...] = jnp.concatenate(
        [(p1 / denom)[:, None], (p2 / denom)[:, None]], axis=1)


def _routing_call(logits):
    t, e = logits.shape
    bm = min(1024, t)
    return pl.pallas_call(
        functools.partial(_routing_body, n_experts=e),
        grid=(t // bm,),
        in_specs=[pl.BlockSpec((bm, e), lambda i: (i, 0))],
        out_specs=[
            pl.BlockSpec((bm, 2), lambda i: (i, 0)),
            pl.BlockSpec((bm, 2), lambda i: (i, 0)),
        ],
        out_shape=[
            jax.ShapeDtypeStruct((t, 2), jnp.int32),
            jax.ShapeDtypeStruct((t, 2), jnp.float32),
        ],
    )(logits)


def _ffn_body(be_ref, x_ref, w1_ref, w2_ref, w3_ref, wrow_ref, out_ref):
    xb16 = x_ref[...]
    dn = (((1,), (1,)), ((), ()))
    h1 = jax.lax.dot_general(xb16, w1_ref[0], dn,
                             preferred_element_type=jnp.float32)
    h3 = jax.lax.dot_general(xb16, w3_ref[0], dn,
                             preferred_element_type=jnp.float32)
    h = (h1 * jax.nn.sigmoid(h1)) * h3
    res = jax.lax.dot_general(h.astype(jnp.bfloat16), w2_ref[0], dn,
                              preferred_element_type=jnp.float32)
    out_ref[...] = res * wrow_ref[...]


def _ffn_call(xs, w1b, w2b, w3b, wsort, block_expert):
    pmax, h = xs.shape
    e, f, _ = w1b.shape
    nb = pmax // _BM
    grid_spec = pltpu.PrefetchScalarGridSpec(
        num_scalar_prefetch=1,
        grid=(nb,),
        in_specs=[
            pl.BlockSpec((_BM, h), lambda i, be: (i, 0)),
            pl.BlockSpec((1, f, h), lambda i, be: (be[i], 0, 0)),
            pl.BlockSpec((1, h, f), lambda i, be: (be[i], 0, 0)),
            pl.BlockSpec((1, f, h), lambda i, be: (be[i], 0, 0)),
            pl.BlockSpec((_BM, 1), lambda i, be: (i, 0)),
        ],
        out_specs=pl.BlockSpec((_BM, h), lambda i, be: (i, 0)),
    )
    return pl.pallas_call(
        _ffn_body,
        grid_spec=grid_spec,
        out_shape=jax.ShapeDtypeStruct((pmax, h), jnp.float32),
        compiler_params=pltpu.CompilerParams(
            dimension_semantics=("arbitrary",),
            vmem_limit_bytes=100 * 1024 * 1024,
        ),
    )(block_expert, xs, w1b, w2b, w3b, wsort)


def _sc_gather_call(x2d, token_idx):
    pmax = token_idx.shape[0]
    h = x2d.shape[1]
    info = plsc.get_sparse_core_info()
    nw = info.num_cores * info.num_subcores
    rpw = pmax // nw
    ch = _GATHER_CHUNK
    nck = rpw // ch
    mesh = plsc.VectorSubcoreMesh(core_axis_name="c", subcore_axis_name="s")

    @functools.partial(
        pl.kernel, mesh=mesh,
        out_type=jax.ShapeDtypeStruct((pmax, h), jnp.int32),
        scratch_types=[
            pltpu.VMEM((ch,), jnp.int32),
            pltpu.VMEM((ch,), jnp.int32),
            pltpu.VMEM((ch, h), jnp.int32),
            pltpu.VMEM((ch, h), jnp.int32),
            pltpu.SemaphoreType.DMA,
            pltpu.SemaphoreType.DMA,
        ],
    )
    def k(x_hbm, idx_hbm, out_hbm, idx_a, idx_b, rows_a, rows_b,
          sem_a, sem_b):
        wid = lax.axis_index("s") * info.num_cores + lax.axis_index("c")
        base = wid * rpw
        ibufs = (idx_a, idx_b)
        rbufs = (rows_a, rows_b)
        sems = (sem_a, sem_b)
        cps = [None, None]
        pltpu.sync_copy(idx_hbm.at[pl.ds(base, ch)], idx_a)
        cps[0] = pltpu.async_copy(x_hbm.at[idx_a], rows_a, sem_a)
        for c in range(1, nck):
            j, pj = c % 2, (c - 1) % 2
            pltpu.sync_copy(idx_hbm.at[pl.ds(base + c * ch, ch)], ibufs[j])
            cps[j] = pltpu.async_copy(x_hbm.at[ibufs[j]], rbufs[j], sems[j])
            cps[pj].wait()
            pltpu.sync_copy(rbufs[pj],
                            out_hbm.at[pl.ds(base + (c - 1) * ch, ch)])
        pj = (nck - 1) % 2
        cps[pj].wait()
        pltpu.sync_copy(rbufs[pj],
                        out_hbm.at[pl.ds(base + (nck - 1) * ch, ch)])

    return k(x2d, token_idx)


def _sc_combine_call(o_sorted, pair_idx):
    two_t = pair_idx.shape[0]
    t = two_t // 2
    h = o_sorted.shape[1]
    info = plsc.get_sparse_core_info()
    nw = info.num_cores * info.num_subcores
    tpw = t // nw
    ct = _COMBINE_CHUNK
    nck = tpw // ct
    mesh = plsc.VectorSubcoreMesh(core_axis_name="c", subcore_axis_name="s")

    @functools.partial(
        pl.kernel, mesh=mesh,
        out_type=jax.ShapeDtypeStruct((t, h), jnp.float32),
        scratch_types=[
            pltpu.VMEM((2 * tpw,), jnp.int32),
            pltpu.VMEM((2 * ct, h), jnp.float32),
            pltpu.VMEM((ct, h), jnp.float32),
            pltpu.SemaphoreType.DMA,
        ],
    )
    def k(o_hbm, pidx_hbm, out_hbm, idx_v, rows_v, acc_v, sem):
        wid = lax.axis_index("s") * info.num_cores + lax.axis_index("c")
        tbase = wid * tpw
        pltpu.sync_copy(pidx_hbm.at[pl.ds(2 * tbase, 2 * tpw)], idx_v)
        for c in range(nck):
            pltpu.async_copy(
                o_hbm.at[idx_v.at[pl.ds(c * 2 * ct, 2 * ct)]], rows_v,
                sem).wait()

            def body(tt, carry):
                for kk in range(h // 16):
                    sl = pl.ds(kk * 16, 16)
                    acc_v[tt, sl] = rows_v[2 * tt, sl] + rows_v[2 * tt + 1, sl]
                return carry

            lax.fori_loop(0, ct, body, 0)
            pltpu.sync_copy(acc_v, out_hbm.at[pl.ds(tbase + c * ct, ct)])

    return k(o_sorted, pair_idx)


def kernel(hidden_states, gate_w, w1, w2, w3):
    b, s, hd = hidden_states.shape
    t = b * s
    e = w1.shape[0]
    x2d = hidden_states.reshape(t, hd)
    router_logits = x2d @ gate_w.T

    sel, tw = _routing_call(router_logits)

    keys = sel.reshape(-1)
    order = jnp.argsort(keys, stable=True).astype(jnp.int32)
    counts = jnp.bincount(keys, length=e)
    padded = ((counts + _BM - 1) // _BM) * _BM
    offs_pad = jnp.concatenate(
        [jnp.zeros((1,), jnp.int32), jnp.cumsum(padded)[:-1].astype(jnp.int32)])
    ks = keys[order]
    first = jnp.searchsorted(ks, jnp.arange(e)).astype(jnp.int32)
    within = jnp.arange(2 * t, dtype=jnp.int32) - first[ks]
    slot = (offs_pad[ks] + within).astype(jnp.int32)

    pmax = 2 * t + e * _BM
    token_idx = jnp.zeros((pmax,), jnp.int32).at[slot].set(order // 2)
    wsort = jnp.zeros((pmax, 1), jnp.float32).at[slot, 0].set(
        tw.reshape(-1)[order])
    pair_idx = jnp.zeros((2 * t,), jnp.int32).at[order].set(slot)
    nb = pmax // _BM
    block_expert = jnp.searchsorted(
        jnp.cumsum(padded), jnp.arange(nb, dtype=jnp.int32) * _BM,
        side="right").astype(jnp.int32).clip(0, e - 1)

    x16_i32 = jax.lax.bitcast_convert_type(
        x2d.astype(jnp.bfloat16).reshape(t, hd // 2, 2), jnp.int32)
    xs_i32 = _sc_gather_call(x16_i32, token_idx)
    xs = jax.lax.bitcast_convert_type(xs_i32, jnp.bfloat16).reshape(-1, hd)
    o_sorted = _ffn_call(
        xs, w1.astype(jnp.bfloat16), w2.astype(jnp.bfloat16),
        w3.astype(jnp.bfloat16), wsort, block_expert)
    out = _sc_combine_call(o_sorted, pair_idx)
    return out.reshape(b, s, hd), router_logits


# speedup vs baseline: 1.2361x; 1.2361x over previous
import functools

import jax
import jax.numpy as jnp
from jax import lax
from jax.experimental import pallas as pl
from jax.experimental.pallas import tpu as pltpu
from jax.experimental.pallas import tpu_sc as plsc

_BM = 256
_GATHER_CHUNK = 40
_COMBINE_CHUNK = 32


def _routing_body(logits_ref, sel_ref, tw_ref, *, n_experts):
    logits = logits_ref[...]
    m = jnp.max(logits, axis=1, keepdims=True)
    p = jnp.exp(logits - m)
    p = p / jnp.sum(p, axis=1, keepdims=True)
    lane = jax.lax.broadcasted_iota(jnp.int32, p.shape, 1)
    e1 = jnp.argmax(p, axis=1).astype(jnp.int32)
    p1 = jnp.max(p, axis=1)
    pm = jnp.where(lane == e1[:, None], -jnp.inf, p)
    e2 = jnp.argmax(pm, axis=1).astype(jnp.int32)
    p2 = jnp.max(pm, axis=1)
    denom = p1 + p2
    sel_ref[...] = jnp.concatenate([e1[:, None], e2[:, None]], axis=1)
    tw_ref[...] = jnp.concatenate(
        [(p1 / denom)[:, None], (p2 / denom)[:, None]], axis=1)


def _routing_call(logits):
    t, e = logits.shape
    bm = min(1024, t)
    return pl.pallas_call(
        functools.partial(_routing_body, n_experts=e),
        grid=(t // bm,),
        in_specs=[pl.BlockSpec((bm, e), lambda i: (i, 0))],
        out_specs=[
            pl.BlockSpec((bm, 2), lambda i: (i, 0)),
            pl.BlockSpec((bm, 2), lambda i: (i, 0)),
        ],
        out_shape=[
            jax.ShapeDtypeStruct((t, 2), jnp.int32),
            jax.ShapeDtypeStruct((t, 2), jnp.float32),
        ],
    )(logits)


def _ffn_body(be_ref, x_ref, w1_ref, w2_ref, w3_ref, wrow_ref, out_ref):
    xb16 = x_ref[...].astype(jnp.bfloat16)
    dn = (((1,), (1,)), ((), ()))
    h1 = jax.lax.dot_general(xb16, w1_ref[0], dn,
                             preferred_element_type=jnp.float32)
    h3 = jax.lax.dot_general(xb16, w3_ref[0], dn,
                             preferred_element_type=jnp.float32)
    h = (h1 * jax.nn.sigmoid(h1)) * h3
    res = jax.lax.dot_general(h.astype(jnp.bfloat16), w2_ref[0], dn,
                              preferred_element_type=jnp.float32)
    out_ref[...] = res * wrow_ref[...]


def _ffn_call(xs, w1b, w2b, w3b, wsort, block_expert):
    pmax, h = xs.shape
    e, f, _ = w1b.shape
    nb = pmax // _BM
    grid_spec = pltpu.PrefetchScalarGridSpec(
        num_scalar_prefetch=1,
        grid=(nb,),
        in_specs=[
            pl.BlockSpec((_BM, h), lambda i, be: (i, 0)),
            pl.BlockSpec((1, f, h), lambda i, be: (be[i], 0, 0)),
            pl.BlockSpec((1, h, f), lambda i, be: (be[i], 0, 0)),
            pl.BlockSpec((1, f, h), lambda i, be: (be[i], 0, 0)),
            pl.BlockSpec((_BM, 1), lambda i, be: (i, 0)),
        ],
        out_specs=pl.BlockSpec((_BM, h), lambda i, be: (i, 0)),
    )
    return pl.pallas_call(
        _ffn_body,
        grid_spec=grid_spec,
        out_shape=jax.ShapeDtypeStruct((pmax, h), jnp.float32),
        compiler_params=pltpu.CompilerParams(
            dimension_semantics=("arbitrary",),
            vmem_limit_bytes=100 * 1024 * 1024,
        ),
    )(block_expert, xs, w1b, w2b, w3b, wsort)


def _sc_gather_call(x2d, token_idx):
    pmax = token_idx.shape[0]
    h = x2d.shape[1]
    info = plsc.get_sparse_core_info()
    nw = info.num_cores * info.num_subcores
    rpw = pmax // nw
    ch = _GATHER_CHUNK
    nck = rpw // ch
    mesh = plsc.VectorSubcoreMesh(core_axis_name="c", subcore_axis_name="s")

    @functools.partial(
        pl.kernel, mesh=mesh,
        out_type=jax.ShapeDtypeStruct((pmax, h), jnp.float32),
        scratch_types=[
            pltpu.VMEM((ch,), jnp.int32),
            pltpu.VMEM((ch,), jnp.int32),
            pltpu.VMEM((ch, h), jnp.float32),
            pltpu.VMEM((ch, h), jnp.float32),
            pltpu.SemaphoreType.DMA,
            pltpu.SemaphoreType.DMA,
        ],
    )
    def k(x_hbm, idx_hbm, out_hbm, idx_a, idx_b, rows_a, rows_b,
          sem_a, sem_b):
        wid = lax.axis_index("s") * info.num_cores + lax.axis_index("c")
        base = wid * rpw
        ibufs = (idx_a, idx_b)
        rbufs = (rows_a, rows_b)
        sems = (sem_a, sem_b)
        cps = [None, None]
        pltpu.sync_copy(idx_hbm.at[pl.ds(base, ch)], idx_a)
        cps[0] = pltpu.async_copy(x_hbm.at[idx_a], rows_a, sem_a)
        for c in range(1, nck):
            j, pj = c % 2, (c - 1) % 2
            pltpu.sync_copy(idx_hbm.at[pl.ds(base + c * ch, ch)], ibufs[j])
            cps[j] = pltpu.async_copy(x_hbm.at[ibufs[j]], rbufs[j], sems[j])
            cps[pj].wait()
            pltpu.sync_copy(rbufs[pj],
                            out_hbm.at[pl.ds(base + (c - 1) * ch, ch)])
        pj = (nck - 1) % 2
        cps[pj].wait()
        pltpu.sync_copy(rbufs[pj],
                        out_hbm.at[pl.ds(base + (nck - 1) * ch, ch)])

    return k(x2d, token_idx)


def _sc_combine_call(o_sorted, pair_idx):
    two_t = pair_idx.shape[0]
    t = two_t // 2
    h = o_sorted.shape[1]
    info = plsc.get_sparse_core_info()
    nw = info.num_cores * info.num_subcores
    tpw = t // nw
    ct = _COMBINE_CHUNK
    nck = tpw // ct
    mesh = plsc.VectorSubcoreMesh(core_axis_name="c", subcore_axis_name="s")

    @functools.partial(
        pl.kernel, mesh=mesh,
        out_type=jax.ShapeDtypeStruct((t, h), jnp.float32),
        scratch_types=[
            pltpu.VMEM((2 * tpw,), jnp.int32),
            pltpu.VMEM((2 * ct, h), jnp.float32),
            pltpu.VMEM((ct, h), jnp.float32),
            pltpu.SemaphoreType.DMA,
        ],
    )
    def k(o_hbm, pidx_hbm, out_hbm, idx_v, rows_v, acc_v, sem):
        wid = lax.axis_index("s") * info.num_cores + lax.axis_index("c")
        tbase = wid * tpw
        pltpu.sync_copy(pidx_hbm.at[pl.ds(2 * tbase, 2 * tpw)], idx_v)
        for c in range(nck):
            pltpu.async_copy(
                o_hbm.at[idx_v.at[pl.ds(c * 2 * ct, 2 * ct)]], rows_v,
                sem).wait()

            def body(tt, carry):
                for kk in range(h // 16):
                    sl = pl.ds(kk * 16, 16)
                    acc_v[tt, sl] = rows_v[2 * tt, sl] + rows_v[2 * tt + 1, sl]
                return carry

            lax.fori_loop(0, ct, body, 0)
            pltpu.sync_copy(acc_v, out_hbm.at[pl.ds(tbase + c * ct, ct)])

    return k(o_sorted, pair_idx)


def kernel(hidden_states, gate_w, w1, w2, w3):
    b, s, hd = hidden_states.shape
    t = b * s
    e = w1.shape[0]
    x2d = hidden_states.reshape(t, hd)
    router_logits = x2d @ gate_w.T

    sel, tw = _routing_call(router_logits)

    keys = sel.reshape(-1)
    order = jnp.argsort(keys, stable=True).astype(jnp.int32)
    counts = jnp.bincount(keys, length=e)
    padded = ((counts + _BM - 1) // _BM) * _BM
    offs_pad = jnp.concatenate(
        [jnp.zeros((1,), jnp.int32), jnp.cumsum(padded)[:-1].astype(jnp.int32)])
    ks = keys[order]
    first = jnp.searchsorted(ks, jnp.arange(e)).astype(jnp.int32)
    within = jnp.arange(2 * t, dtype=jnp.int32) - first[ks]
    slot = (offs_pad[ks] + within).astype(jnp.int32)

    pmax = 2 * t + e * _BM
    token_idx = jnp.zeros((pmax,), jnp.int32).at[slot].set(order // 2)
    wsort = jnp.zeros((pmax, 1), jnp.float32).at[slot, 0].set(
        tw.reshape(-1)[order])
    pair_idx = jnp.zeros((2 * t,), jnp.int32).at[order].set(slot)
    nb = pmax // _BM
    block_expert = jnp.searchsorted(
        jnp.cumsum(padded), jnp.arange(nb, dtype=jnp.int32) * _BM,
        side="right").astype(jnp.int32).clip(0, e - 1)

    w1b = w1.astype(jnp.bfloat16)
    w2b = w2.astype(jnp.bfloat16)
    w3b = w3.astype(jnp.bfloat16)
    half = pmax // 2
    gather_grain = 32 * _GATHER_CHUNK
    nsplit = 2 if (half % _BM == 0 and half % gather_grain == 0) else 1
    half = pmax // nsplit
    nbh = half // _BM
    xs_halves = [
        _sc_gather_call(x2d, token_idx[i * half:(i + 1) * half])
        for i in range(nsplit)
    ]
    o_halves = [
        _ffn_call(xs_halves[i], w1b, w2b, w3b,
                  wsort[i * half:(i + 1) * half],
                  block_expert[i * nbh:(i + 1) * nbh])
        for i in range(nsplit)
    ]
    o_sorted = (jnp.concatenate(o_halves, axis=0) if nsplit > 1
                else o_halves[0])
    out = _sc_combine_call(o_sorted, pair_idx)
    return out.reshape(b, s, hd), router_logits


# speedup vs baseline: 1.3960x; 1.1294x over previous
import functools

import jax
import jax.numpy as jnp
from jax import lax
from jax.experimental import pallas as pl
from jax.experimental.pallas import tpu as pltpu
from jax.experimental.pallas import tpu_sc as plsc

_BM = 256
_GATHER_CHUNK = 40
_COMBINE_CHUNK = 16


def _routing_body(logits_ref, sel_ref, tw_ref, *, n_experts):
    logits = logits_ref[...]
    m = jnp.max(logits, axis=1, keepdims=True)
    p = jnp.exp(logits - m)
    p = p / jnp.sum(p, axis=1, keepdims=True)
    lane = jax.lax.broadcasted_iota(jnp.int32, p.shape, 1)
    e1 = jnp.argmax(p, axis=1).astype(jnp.int32)
    p1 = jnp.max(p, axis=1)
    pm = jnp.where(lane == e1[:, None], -jnp.inf, p)
    e2 = jnp.argmax(pm, axis=1).astype(jnp.int32)
    p2 = jnp.max(pm, axis=1)
    denom = p1 + p2
    sel_ref[...] = jnp.concatenate([e1[:, None], e2[:, None]], axis=1)
    tw_ref[...] = jnp.concatenate(
        [(p1 / denom)[:, None], (p2 / denom)[:, None]], axis=1)


def _routing_call(logits):
    t, e = logits.shape
    bm = min(1024, t)
    return pl.pallas_call(
        functools.partial(_routing_body, n_experts=e),
        grid=(t // bm,),
        in_specs=[pl.BlockSpec((bm, e), lambda i: (i, 0))],
        out_specs=[
            pl.BlockSpec((bm, 2), lambda i: (i, 0)),
            pl.BlockSpec((bm, 2), lambda i: (i, 0)),
        ],
        out_shape=[
            jax.ShapeDtypeStruct((t, 2), jnp.int32),
            jax.ShapeDtypeStruct((t, 2), jnp.float32),
        ],
    )(logits)


def _ffn_body(be_ref, x_ref, w1_ref, w2_ref, w3_ref, wrow_ref, out_ref):
    xb16 = x_ref[...].astype(jnp.bfloat16)
    dn = (((1,), (1,)), ((), ()))
    h1 = jax.lax.dot_general(xb16, w1_ref[0], dn,
                             preferred_element_type=jnp.float32)
    h3 = jax.lax.dot_general(xb16, w3_ref[0], dn,
                             preferred_element_type=jnp.float32)
    h = (h1 * jax.nn.sigmoid(h1)) * h3
    res = jax.lax.dot_general(h.astype(jnp.bfloat16), w2_ref[0], dn,
                              preferred_element_type=jnp.float32)
    out_ref[...] = res * wrow_ref[...]


def _ffn_call(xs, w1b, w2b, w3b, wsort, block_expert):
    pmax, h = xs.shape
    e, f, _ = w1b.shape
    nb = pmax // _BM
    grid_spec = pltpu.PrefetchScalarGridSpec(
        num_scalar_prefetch=1,
        grid=(nb,),
        in_specs=[
            pl.BlockSpec((_BM, h), lambda i, be: (i, 0)),
            pl.BlockSpec((1, f, h), lambda i, be: (be[i], 0, 0)),
            pl.BlockSpec((1, h, f), lambda i, be: (be[i], 0, 0)),
            pl.BlockSpec((1, f, h), lambda i, be: (be[i], 0, 0)),
            pl.BlockSpec((_BM, 1), lambda i, be: (i, 0)),
        ],
        out_specs=pl.BlockSpec((_BM, h), lambda i, be: (i, 0)),
    )
    return pl.pallas_call(
        _ffn_body,
        grid_spec=grid_spec,
        out_shape=jax.ShapeDtypeStruct((pmax, h), jnp.float32),
        compiler_params=pltpu.CompilerParams(
            dimension_semantics=("arbitrary",),
            vmem_limit_bytes=100 * 1024 * 1024,
        ),
    )(block_expert, xs, w1b, w2b, w3b, wsort)


def _sc_gather_call(x2d, token_idx):
    pmax = token_idx.shape[0]
    h = x2d.shape[1]
    info = plsc.get_sparse_core_info()
    nw = info.num_cores * info.num_subcores
    rpw = pmax // nw
    ch = _GATHER_CHUNK
    nck = rpw // ch
    mesh = plsc.VectorSubcoreMesh(core_axis_name="c", subcore_axis_name="s")

    @functools.partial(
        pl.kernel, mesh=mesh,
        out_type=jax.ShapeDtypeStruct((pmax, h), jnp.float32),
        scratch_types=[
            pltpu.VMEM((ch,), jnp.int32),
            pltpu.VMEM((ch,), jnp.int32),
            pltpu.VMEM((ch, h), jnp.float32),
            pltpu.VMEM((ch, h), jnp.float32),
            pltpu.SemaphoreType.DMA,
            pltpu.SemaphoreType.DMA,
        ],
    )
    def k(x_hbm, idx_hbm, out_hbm, idx_a, idx_b, rows_a, rows_b,
          sem_a, sem_b):
        wid = lax.axis_index("s") * info.num_cores + lax.axis_index("c")
        base = wid * rpw
        ibufs = (idx_a, idx_b)
        rbufs = (rows_a, rows_b)
        sems = (sem_a, sem_b)
        cps = [None, None]
        pltpu.sync_copy(idx_hbm.at[pl.ds(base, ch)], idx_a)
        cps[0] = pltpu.async_copy(x_hbm.at[idx_a], rows_a, sem_a)
        for c in range(1, nck):
            j, pj = c % 2, (c - 1) % 2
            pltpu.sync_copy(idx_hbm.at[pl.ds(base + c * ch, ch)], ibufs[j])
            cps[j] = pltpu.async_copy(x_hbm.at[ibufs[j]], rbufs[j], sems[j])
            cps[pj].wait()
            pltpu.sync_copy(rbufs[pj],
                            out_hbm.at[pl.ds(base + (c - 1) * ch, ch)])
        pj = (nck - 1) % 2
        cps[pj].wait()
        pltpu.sync_copy(rbufs[pj],
                        out_hbm.at[pl.ds(base + (nck - 1) * ch, ch)])

    return k(x2d, token_idx)


def _sc_combine_call(o_sorted, pair_idx):
    two_t = pair_idx.shape[0]
    t = two_t // 2
    h = o_sorted.shape[1]
    info = plsc.get_sparse_core_info()
    nw = info.num_cores * info.num_subcores
    tpw = t // nw
    ct = _COMBINE_CHUNK
    nck = tpw // ct
    mesh = plsc.VectorSubcoreMesh(core_axis_name="c", subcore_axis_name="s")

    @functools.partial(
        pl.kernel, mesh=mesh,
        out_type=jax.ShapeDtypeStruct((t, h), jnp.float32),
        scratch_types=[
            pltpu.VMEM((2 * ct,), jnp.int32),
            pltpu.VMEM((2 * ct,), jnp.int32),
            pltpu.VMEM((2 * ct, h), jnp.float32),
            pltpu.VMEM((2 * ct, h), jnp.float32),
            pltpu.VMEM((ct, h), jnp.float32),
            pltpu.SemaphoreType.DMA,
            pltpu.SemaphoreType.DMA,
        ],
    )
    def k(o_hbm, pidx_hbm, out_hbm, idx_a, idx_b, rows_a, rows_b, acc_v,
          sem_a, sem_b):
        wid = lax.axis_index("s") * info.num_cores + lax.axis_index("c")
        tbase = wid * tpw
        ibufs = (idx_a, idx_b)
        rbufs = (rows_a, rows_b)
        sems = (sem_a, sem_b)
        cps = [None, None]

        def start(c):
            j = c % 2
            pltpu.sync_copy(
                pidx_hbm.at[pl.ds(2 * tbase + c * 2 * ct, 2 * ct)], ibufs[j])
            cps[j] = pltpu.async_copy(o_hbm.at[ibufs[j]], rbufs[j], sems[j])

        start(0)
        for c in range(nck):
            j = c % 2
            if c + 1 < nck:
                start(c + 1)
            cps[j].wait()
            rows_v = rbufs[j]

            def body(tt, carry, rows_v=rows_v):
                for kk in range(h // 16):
                    sl = pl.ds(kk * 16, 16)
                    acc_v[tt, sl] = rows_v[2 * tt, sl] + rows_v[2 * tt + 1, sl]
                return carry

            lax.fori_loop(0, ct, body, 0)
            pltpu.sync_copy(acc_v, out_hbm.at[pl.ds(tbase + c * ct, ct)])

    return k(o_sorted, pair_idx)


def kernel(hidden_states, gate_w, w1, w2, w3):
    b, s, hd = hidden_states.shape
    t = b * s
    e = w1.shape[0]
    x2d = hidden_states.reshape(t, hd)
    router_logits = x2d @ gate_w.T

    sel, tw = _routing_call(router_logits)

    keys = sel.reshape(-1)
    order = jnp.argsort(keys, stable=True).astype(jnp.int32)
    erange = jnp.arange(e, dtype=jnp.int32)
    counts = jnp.sum((keys[None, :] == erange[:, None]),
                     axis=1).astype(jnp.int32)
    padded = ((counts + _BM - 1) // _BM) * _BM
    cum = jnp.cumsum(padded).astype(jnp.int32)
    offs_pad = cum - padded
    csum_counts = jnp.cumsum(counts).astype(jnp.int32)
    first = csum_counts - counts
    ks = keys[order]
    j_arange = jnp.arange(2 * t, dtype=jnp.int32)
    slot_sorted = offs_pad[ks] + (j_arange - first[ks])
    inv_order = jnp.argsort(order).astype(jnp.int32)
    pair_idx = slot_sorted[inv_order]

    pmax = 2 * t + e * _BM
    q = jnp.arange(pmax, dtype=jnp.int32)
    eq = jnp.minimum(
        jnp.sum((q[:, None] >= cum[None, :]), axis=1).astype(jnp.int32),
        e - 1)
    rq = q - offs_pad[eq]
    valid = rq < counts[eq]
    jq = jnp.clip(first[eq] + jnp.minimum(rq, counts[eq] - 1), 0, 2 * t - 1)
    src_pair = order[jq]
    token_idx = jnp.where(valid, src_pair // 2, 0).astype(jnp.int32)
    wsort = jnp.where(valid, tw.reshape(-1)[src_pair], 0.0)[:, None]
    nb = pmax // _BM
    qb = jnp.arange(nb, dtype=jnp.int32) * _BM
    block_expert = jnp.minimum(
        jnp.sum((qb[:, None] >= cum[None, :]), axis=1).astype(jnp.int32),
        e - 1)

    w1b = w1.astype(jnp.bfloat16)
    w2b = w2.astype(jnp.bfloat16)
    w3b = w3.astype(jnp.bfloat16)
    xs = _sc_gather_call(x2d, token_idx)
    o_sorted = _ffn_call(xs, w1b, w2b, w3b, wsort, block_expert)
    out = _sc_combine_call(o_sorted, pair_idx)
    return out.reshape(b, s, hd), router_logits
